# baseline (speedup 1.0000x reference)
import functools

import jax
import jax.numpy as jnp
from jax import lax
from jax.experimental import pallas as pl
from jax.experimental.pallas import tpu as pltpu
from jax.experimental.pallas import tpu_sc as plsc

NC = 2
NS = 16
NW = NC * NS
L = 16

_mesh = functools.partial(
    plsc.VectorSubcoreMesh,
    core_axis_name="c",
    subcore_axis_name="s",
    num_cores=NC,
    num_subcores=NS,
)

_params = pltpu.CompilerParams(needs_layout_passes=False)


def _worker_id():
    return lax.axis_index("s") * NC + lax.axis_index("c")


def _d2_body(n_per_w, chunk, pd_hbm, out_hbm, buf, accbuf, sem):
    wid = _worker_id()
    base = wid * n_per_w
    n_chunks = n_per_w // chunk

    def issue(i, b):
        pltpu.async_copy(
            pd_hbm.at[pl.ds(base + i * chunk, chunk)],
            buf.at[pl.ds(b * chunk, chunk)], sem)

    issue(0, 0)

    def chunk_body(i, acc):
        b = jnp.bitwise_and(i, 1)
        pltpu.make_async_copy(
            pd_hbm.at[pl.ds(base + i * chunk, chunk)],
            buf.at[pl.ds(b * chunk, chunk)], sem).wait()

        @pl.when(i + 1 < n_chunks)
        def _():
            issue(i + 1, 1 - b)

        boff = b * chunk

        @plsc.parallel_loop(0, chunk, L, unroll=5, carry=acc)
        def vec_body(off, a):
            v = buf[pl.ds(boff + off, L)]
            return a + v * v

        return vec_body

    acc = lax.fori_loop(0, n_chunks, chunk_body, jnp.zeros((L,), jnp.float32))
    accbuf[...] = acc
    pltpu.sync_copy(accbuf, out_hbm.at[pl.ds(wid * L, L)])


def _lj_body(n_per_w, chunk, pd_hbm, fi_hbm, sig_hbm, eps_hbm, part_hbm,
             en_hbm, fo_hbm,
             sig_v, eps_v, part_v, red_v, pd_v, fi_v, en_v, fo_v,
             sem_in, sem_out):
    wid = _worker_id()
    base = wid * n_per_w
    n_chunks = n_per_w // chunk

    pltpu.sync_copy(sig_hbm, sig_v)
    pltpu.sync_copy(eps_hbm, eps_v)
    pltpu.sync_copy(part_hbm, part_v)

    def pr_body(i, a):
        return a + part_v[pl.ds(i * L, L)]

    accv = lax.fori_loop(0, NW, pr_body, jnp.zeros((L,), jnp.float32))

    iota = jnp.arange(L, dtype=jnp.int32)
    for step in (8, 4, 2, 1):
        red_v[...] = accv
        accv = accv + plsc.load_gather(red_v, [jnp.bitwise_xor(iota, step)])
    d2 = accv
    invd2 = 1.0 / d2
    invd6 = invd2 * invd2 * invd2
    c24 = 24.0 * invd2

    iota2 = iota * 2

    cw = chunk // 2
    base_w = wid * (n_per_w // 2)

    def issue_in(i, b):
        cb = base + i * chunk
        pltpu.async_copy(pd_hbm.at[pl.ds(cb, chunk)],
                         pd_v.at[pl.ds(b * chunk, chunk)], sem_in)
        pltpu.async_copy(fi_hbm.at[pl.ds(base_w + i * cw, cw)],
                         fi_v.at[pl.ds(b * cw, cw)], sem_in)

    def wait_in(i, b):
        cb = base + i * chunk
        pltpu.make_async_copy(pd_hbm.at[pl.ds(cb, chunk)],
                              pd_v.at[pl.ds(b * chunk, chunk)], sem_in).wait()
        pltpu.make_async_copy(fi_hbm.at[pl.ds(base_w + i * cw, cw)],
                              fi_v.at[pl.ds(b * cw, cw)], sem_in).wait()

    def wait_out(b):
        bo = b * chunk
        pltpu.make_async_copy(en_v.at[pl.ds(bo, chunk)],
                              en_hbm.at[pl.ds(base, chunk)], sem_out).wait()
        pltpu.make_async_copy(fo_v.at[pl.ds(bo, chunk)],
                              fo_hbm.at[pl.ds(base, chunk)], sem_out).wait()

    def lj_halves(s, e, pd):
        r = s / pd
        r2 = r * r
        r4 = r2 * r2
        r6 = r4 * r2
        r12 = r6 * r6
        en = (r12 - r6) * (e * 4.0)
        s2 = s * s
        s4 = s2 * s2
        s6 = s4 * s2
        q6 = s6 * invd6
        q12 = q6 * q6
        fo = (q12 + q12 - q6) * (e * pd) * c24
        return en, fo

    issue_in(0, 0)

    def chunk_body(i, _):
        b = jnp.bitwise_and(i, 1)
        cb = base + i * chunk
        bo = b * chunk
        wait_in(i, b)

        @pl.when(i + 1 < n_chunks)
        def _():
            issue_in(i + 1, 1 - b)

        @pl.when(i >= 2)
        def _():
            wait_out(b)

        bo_w = b * cw

        @plsc.parallel_loop(0, cw, L, unroll=5)
        def vec_body(off_w):
            fiw = fi_v[pl.ds(bo_w + off_w, L)]
            fi_e = jnp.bitwise_and(fiw, 0xFFFF)
            fi_o = lax.shift_right_logical(fiw, 16)
            pos_e = iota2 + (bo + off_w * 2)
            pos_o = pos_e + 1
            pd_e = plsc.load_gather(pd_v, [pos_e])
            pd_o = plsc.load_gather(pd_v, [pos_o])
            s_e = plsc.load_gather(sig_v, [fi_e])
            s_o = plsc.load_gather(sig_v, [fi_o])
            e_e = plsc.load_gather(eps_v, [fi_e])
            e_o = plsc.load_gather(eps_v, [fi_o])
            en_e, fo_e = lj_halves(s_e, e_e, pd_e)
            en_o, fo_o = lj_halves(s_o, e_o, pd_o)
            plsc.store_scatter(en_v, [pos_e], en_e)
            plsc.store_scatter(en_v, [pos_o], en_o)
            plsc.store_scatter(fo_v, [pos_e], fo_e)
            plsc.store_scatter(fo_v, [pos_o], fo_o)

        pltpu.async_copy(en_v.at[pl.ds(bo, chunk)],
                         en_hbm.at[pl.ds(cb, chunk)], sem_out)
        pltpu.async_copy(fo_v.at[pl.ds(bo, chunk)],
                         fo_hbm.at[pl.ds(cb, chunk)], sem_out)
        return 0

    lax.fori_loop(0, n_chunks, chunk_body, 0)
    wait_out(jnp.int32(0))
    wait_out(jnp.int32(1))


def kernel(pair_diff, atom_types, sig, eps):
    n = pair_diff.shape[0]
    n_types = sig.shape[0]
    assert n % (NW * 2 * L) == 0
    n_per_w = n // NW

    chunk1 = 50000
    assert n_per_w % chunk1 == 0 and chunk1 % L == 0

    d2_k = pl.kernel(
        functools.partial(_d2_body, n_per_w, chunk1),
        out_type=jax.ShapeDtypeStruct((NW * L,), jnp.float32),
        mesh=_mesh(),
        compiler_params=_params,
        scratch_types=[
            pltpu.VMEM((2 * chunk1,), jnp.float32),
            pltpu.VMEM((L,), jnp.float32),
            pltpu.SemaphoreType.DMA,
        ],
    )
    partials = d2_k(pair_diff)

    fi = atom_types[:, 0] * n_types + atom_types[:, 1]
    flat_idx = jnp.bitwise_or(fi[0::2], fi[1::2] << 16)

    chunk2 = 8000
    assert n_per_w % chunk2 == 0 and chunk2 % (2 * L) == 0

    lj_k = pl.kernel(
        functools.partial(_lj_body, n_per_w, chunk2),
        out_type=(
            jax.ShapeDtypeStruct((n,), jnp.float32),
            jax.ShapeDtypeStruct((n,), jnp.float32),
        ),
        mesh=_mesh(),
        compiler_params=_params,
        scratch_types=[
            pltpu.VMEM((n_types * n_types,), jnp.float32),
            pltpu.VMEM((n_types * n_types,), jnp.float32),
            pltpu.VMEM((NW * L,), jnp.float32),
            pltpu.VMEM((L,), jnp.float32),
            pltpu.VMEM((2 * chunk2,), jnp.float32),
            pltpu.VMEM((chunk2,), jnp.int32),
            pltpu.VMEM((2 * chunk2,), jnp.float32),
            pltpu.VMEM((2 * chunk2,), jnp.float32),
            pltpu.SemaphoreType.DMA,
            pltpu.SemaphoreType.DMA,
        ],
    )
    energy, forces = lj_k(
        pair_diff, flat_idx, sig.reshape(-1), eps.reshape(-1), partials)
    return (energy, forces)


# speedup vs baseline: 9.1893x; 9.1893x over previous
import functools

import jax
import jax.numpy as jnp
from jax import lax
from jax.experimental import pallas as pl
from jax.experimental.pallas import tpu as pltpu
from jax.experimental.pallas import tpu_sc as plsc

NC = 2
NS = 16
NW = NC * NS
L = 16

_mesh = functools.partial(
    plsc.VectorSubcoreMesh,
    core_axis_name="c",
    subcore_axis_name="s",
    num_cores=NC,
    num_subcores=NS,
)

_params = pltpu.CompilerParams(needs_layout_passes=False)


def _worker_id():
    return lax.axis_index("s") * NC + lax.axis_index("c")


def _d2_body(n_per_w, chunk, pd_hbm, out_hbm, buf, accbuf, sem):
    wid = _worker_id()
    base = wid * n_per_w
    n_chunks = n_per_w // chunk

    def issue(i, b):
        pltpu.async_copy(
            pd_hbm.at[pl.ds(base + i * chunk, chunk)],
            buf.at[pl.ds(b * chunk, chunk)], sem)

    issue(0, 0)

    def chunk_body(i, acc):
        b = jnp.bitwise_and(i, 1)
        pltpu.make_async_copy(
            pd_hbm.at[pl.ds(base + i * chunk, chunk)],
            buf.at[pl.ds(b * chunk, chunk)], sem).wait()

        @pl.when(i + 1 < n_chunks)
        def _():
            issue(i + 1, 1 - b)

        boff = b * chunk

        @plsc.parallel_loop(0, chunk, L, unroll=5, carry=acc)
        def vec_body(off, a):
            v = buf[pl.ds(boff + off, L)]
            return a + v * v

        return vec_body

    acc = lax.fori_loop(0, n_chunks, chunk_body, jnp.zeros((L,), jnp.float32))
    accbuf[...] = acc
    pltpu.sync_copy(accbuf, out_hbm.at[pl.ds(wid * L, L)])


def _lj_body(n_per_w, chunk, pd_hbm, fi_hbm, sig_hbm, eps_hbm, part_hbm,
             en_hbm, fo_hbm,
             sig_v, eps_v, part_v, red_v, pd_v, fi_v, en_v, fo_v,
             sem_in, sem_out):
    wid = _worker_id()
    base = wid * n_per_w
    n_chunks = n_per_w // chunk

    pltpu.sync_copy(sig_hbm, sig_v)
    pltpu.sync_copy(eps_hbm, eps_v)
    pltpu.sync_copy(part_hbm, part_v)

    def pr_body(i, a):
        return a + part_v[pl.ds(i * L, L)]

    accv = lax.fori_loop(0, NW, pr_body, jnp.zeros((L,), jnp.float32))

    iota = jnp.arange(L, dtype=jnp.int32)
    for step in (8, 4, 2, 1):
        red_v[...] = accv
        accv = accv + plsc.load_gather(red_v, [jnp.bitwise_xor(iota, step)])
    d2 = accv
    invd2 = 1.0 / d2
    invd6 = invd2 * invd2 * invd2
    c24 = 24.0 * invd2

    def issue_in(i, b):
        cb = base + i * chunk
        bo = b * chunk
        pltpu.async_copy(pd_hbm.at[pl.ds(cb, chunk)],
                         pd_v.at[pl.ds(bo, chunk)], sem_in)
        pltpu.async_copy(fi_hbm.at[pl.ds(cb, chunk)],
                         fi_v.at[pl.ds(bo, chunk)], sem_in)

    def wait_in(i, b):
        cb = base + i * chunk
        bo = b * chunk
        pltpu.make_async_copy(pd_hbm.at[pl.ds(cb, chunk)],
                              pd_v.at[pl.ds(bo, chunk)], sem_in).wait()
        pltpu.make_async_copy(fi_hbm.at[pl.ds(cb, chunk)],
                              fi_v.at[pl.ds(bo, chunk)], sem_in).wait()

    def wait_out(b):
        bo = b * chunk
        pltpu.make_async_copy(en_v.at[pl.ds(bo, chunk)],
                              en_hbm.at[pl.ds(base, chunk)], sem_out).wait()
        pltpu.make_async_copy(fo_v.at[pl.ds(bo, chunk)],
                              fo_hbm.at[pl.ds(base, chunk)], sem_out).wait()

    def lj_one(s, e, pd):
        r = s / pd
        r2 = r * r
        r4 = r2 * r2
        r6 = r4 * r2
        r12 = r6 * r6
        en = (r12 - r6) * (e * 4.0)
        s2 = s * s
        s4 = s2 * s2
        s6 = s4 * s2
        q6 = s6 * invd6
        q12 = q6 * q6
        fo = (q12 + q12 - q6) * (e * pd) * c24
        return en, fo

    issue_in(0, 0)

    def chunk_body(i, _):
        b = jnp.bitwise_and(i, 1)
        cb = base + i * chunk
        bo = b * chunk
        wait_in(i, b)

        @pl.when(i + 1 < n_chunks)
        def _():
            issue_in(i + 1, 1 - b)

        @pl.when(i >= 2)
        def _():
            wait_out(b)

        @plsc.parallel_loop(0, chunk, L, unroll=5)
        def vec_body(off):
            fi = fi_v[pl.ds(bo + off, L)]
            s = plsc.load_gather(sig_v, [fi])
            e = plsc.load_gather(eps_v, [fi])
            pd = pd_v[pl.ds(bo + off, L)]
            en, fo = lj_one(s, e, pd)
            en_v[pl.ds(bo + off, L)] = en
            fo_v[pl.ds(bo + off, L)] = fo

        pltpu.async_copy(en_v.at[pl.ds(bo, chunk)],
                         en_hbm.at[pl.ds(cb, chunk)], sem_out)
        pltpu.async_copy(fo_v.at[pl.ds(bo, chunk)],
                         fo_hbm.at[pl.ds(cb, chunk)], sem_out)
        return 0

    lax.fori_loop(0, n_chunks, chunk_body, 0)
    wait_out(jnp.int32(0))
    wait_out(jnp.int32(1))


def kernel(pair_diff, atom_types, sig, eps):
    n = pair_diff.shape[0]
    n_types = sig.shape[0]
    assert n % (NW * 2 * L) == 0
    n_per_w = n // NW

    chunk1 = 50000
    assert n_per_w % chunk1 == 0 and chunk1 % L == 0

    d2_k = pl.kernel(
        functools.partial(_d2_body, n_per_w, chunk1),
        out_type=jax.ShapeDtypeStruct((NW * L,), jnp.float32),
        mesh=_mesh(),
        compiler_params=_params,
        scratch_types=[
            pltpu.VMEM((2 * chunk1,), jnp.float32),
            pltpu.VMEM((L,), jnp.float32),
            pltpu.SemaphoreType.DMA,
        ],
    )
    partials = d2_k(pair_diff)

    flat_idx = atom_types[:, 0] * n_types + atom_types[:, 1]

    chunk2 = 8000
    assert n_per_w % chunk2 == 0 and chunk2 % L == 0

    lj_k = pl.kernel(
        functools.partial(_lj_body, n_per_w, chunk2),
        out_type=(
            jax.ShapeDtypeStruct((n,), jnp.float32),
            jax.ShapeDtypeStruct((n,), jnp.float32),
        ),
        mesh=_mesh(),
        compiler_params=_params,
        scratch_types=[
            pltpu.VMEM((n_types * n_types,), jnp.float32),
            pltpu.VMEM((n_types * n_types,), jnp.float32),
            pltpu.VMEM((NW * L,), jnp.float32),
            pltpu.VMEM((L,), jnp.float32),
            pltpu.VMEM((2 * chunk2,), jnp.float32),
            pltpu.VMEM((2 * chunk2,), jnp.int32),
            pltpu.VMEM((2 * chunk2,), jnp.float32),
            pltpu.VMEM((2 * chunk2,), jnp.float32),
            pltpu.SemaphoreType.DMA,
            pltpu.SemaphoreType.DMA,
        ],
    )
    energy, forces = lj_k(
        pair_diff, flat_idx, sig.reshape(-1), eps.reshape(-1), partials)
    return (energy, forces)
